# Initial kernel scaffold; baseline (speedup 1.0000x reference)
import jax
import jax.numpy as jnp
from jax.experimental import pallas as pl


def kernel(x, edge_index, W0, b0, W1, b1, W2, b2, g1, be1, g2, be2):
    raise NotImplementedError("write your pallas kernel here")


# speedup vs baseline: 4.7341x; 4.7341x over previous
import functools

import jax
import jax.numpy as jnp
from jax.experimental import pallas as pl
from jax.experimental.pallas import tpu as pltpu

_N = 10000
_E = 320000
_D = 128
_DOUT = 40
_NP = 10240
_TH = 0.1

_BD = 1024
_BS = 512
_PB = 512

_HI = jax.lax.Precision.HIGHEST


def _prep_body(x_ref, w_ref, h_ref, n_ref):
    x = x_ref[...]
    h_ref[...] = jax.lax.dot_general(
        x, w_ref[...], (((1,), (1,)), ((), ())),
        preferred_element_type=jnp.float32, precision=_HI)
    n_ref[...] = jnp.sqrt(jnp.sum(x * x, axis=1, keepdims=True))


def _prep(x, w, *, np_=_NP, pb=_PB, interpret=False):
    return pl.pallas_call(
        _prep_body,
        grid=(np_ // pb,),
        in_specs=[
            pl.BlockSpec((pb, _D), lambda i: (i, 0)),
            pl.BlockSpec((_D, _D), lambda i: (0, 0)),
        ],
        out_specs=[
            pl.BlockSpec((pb, _D), lambda i: (i, 0)),
            pl.BlockSpec((pb, 1), lambda i: (i, 0)),
        ],
        out_shape=[
            jax.ShapeDtypeStruct((np_, _D), jnp.float32),
            jax.ShapeDtypeStruct((np_, 1), jnp.float32),
        ],
        interpret=interpret,
    )(x, w)


def _att_body(mode, xr_ref, xc_ref, h_ref, c_ref, ni_ref, njt_ref,
              b_ref, g_ref, be_ref, y_ref, acc_ref, z_ref):
    j = pl.program_id(1)
    nj = pl.num_programs(1)

    @pl.when(j == 0)
    def _():
        acc_ref[...] = jnp.zeros_like(acc_ref)
        z_ref[...] = jnp.zeros_like(z_ref)

    s = jax.lax.dot_general(
        xr_ref[...], xc_ref[...], (((1,), (1,)), ((), ())),
        preferred_element_type=jnp.float32, precision=_HI)
    den = jnp.maximum(ni_ref[...] * njt_ref[...], 1e-8)
    cos = s / den
    a = jnp.where(cos < _TH, 1e-6, cos)
    w = a * c_ref[...].astype(jnp.float32)
    z_ref[...] += jnp.sum(w, axis=1, keepdims=True)
    acc_ref[...] += jax.lax.dot_general(
        w, h_ref[...], (((1,), (0,)), ((), ())),
        preferred_element_type=jnp.float32, precision=_HI)

    @pl.when(j == nj - 1)
    def _():
        z = z_ref[...]
        v = acc_ref[...] / z + b_ref[...]
        if mode == "ln":
            mu = jnp.mean(v, axis=1, keepdims=True)
            var = jnp.mean((v - mu) ** 2, axis=1, keepdims=True)
            yv = (v - mu) / jnp.sqrt(var + 1e-5) * g_ref[...] + be_ref[...]
            yv = jnp.maximum(yv, 0.0)
            y_ref[...] = jnp.where(z > 0.0, yv, 0.0)
        else:
            col = jax.lax.broadcasted_iota(jnp.int32, v.shape, 1)
            vm = jnp.where(col < _DOUT, v, jnp.float32(-1e30))
            m = jnp.max(vm, axis=1, keepdims=True)
            e = jnp.where(col < _DOUT, jnp.exp(vm - m), 0.0)
            lse = jnp.log(jnp.sum(e, axis=1, keepdims=True))
            y_ref[...] = vm - m - lse


def _att(x, h, c, n, nt, b, g, be, mode, *, np_=_NP, bd=_BD, bs=_BS,
         interpret=False):
    body = functools.partial(_att_body, mode)
    return pl.pallas_call(
        body,
        grid=(np_ // bd, np_ // bs),
        in_specs=[
            pl.BlockSpec((bd, _D), lambda i, j: (i, 0)),
            pl.BlockSpec((bs, _D), lambda i, j: (j, 0)),
            pl.BlockSpec((bs, _D), lambda i, j: (j, 0)),
            pl.BlockSpec((bd, bs), lambda i, j: (i, j)),
            pl.BlockSpec((bd, 1), lambda i, j: (i, 0)),
            pl.BlockSpec((1, bs), lambda i, j: (0, j)),
            pl.BlockSpec((1, _D), lambda i, j: (0, 0)),
            pl.BlockSpec((1, _D), lambda i, j: (0, 0)),
            pl.BlockSpec((1, _D), lambda i, j: (0, 0)),
        ],
        out_specs=pl.BlockSpec((bd, _D), lambda i, j: (i, 0)),
        out_shape=jax.ShapeDtypeStruct((np_, _D), jnp.float32),
        scratch_shapes=[
            pltpu.VMEM((bd, _D), jnp.float32),
            pltpu.VMEM((bd, 1), jnp.float32),
        ],
        compiler_params=pltpu.CompilerParams(
            dimension_semantics=("parallel", "arbitrary")),
        interpret=interpret,
    )(x, x, h, c, n, nt, b, g, be)


def _build_count_matrix(edge_index, *, n=_N, np_=_NP):
    s0 = edge_index[0]
    d0 = edge_index[1]
    self_m = s0 == d0
    s1 = jnp.where(self_m, n, s0)
    d1 = jnp.where(self_m, n, d0)
    flat = d1 * np_ + s1
    loops = jnp.arange(n, dtype=jnp.int32) * (np_ + 1)
    all_flat = jnp.concatenate([flat, loops])
    c = jnp.zeros((np_ * np_,), jnp.int8).at[all_flat].add(1)
    return c.reshape(np_, np_)


def kernel(x, edge_index, W0, b0, W1, b1, W2, b2, g1, be1, g2, be2):
    c = _build_count_matrix(edge_index)
    xp = jnp.pad(x, ((0, _NP - _N), (0, 0)))

    w2p = jnp.zeros((_D, _D), jnp.float32).at[:_DOUT].set(W2)
    b2p = jnp.zeros((1, _D), jnp.float32).at[0, :_DOUT].set(b2)
    ones = jnp.ones((1, _D), jnp.float32)
    zeros = jnp.zeros((1, _D), jnp.float32)

    h0, n0 = _prep(xp, W0)
    x1 = _att(xp, h0, c, n0, n0.reshape(1, _NP), b0.reshape(1, _D),
              g1.reshape(1, _D), be1.reshape(1, _D), "ln")
    h1, n1 = _prep(x1, W1)
    x2 = _att(x1, h1, c, n1, n1.reshape(1, _NP), b1.reshape(1, _D),
              g2.reshape(1, _D), be2.reshape(1, _D), "ln")
    h2, n2 = _prep(x2, w2p)
    y = _att(x2, h2, c, n2, n2.reshape(1, _NP), b2p, ones, zeros, "lsm")
    return y[:_N, :_DOUT]
